# Initial kernel scaffold; baseline (speedup 1.0000x reference)
import functools

import jax
import jax.numpy as jnp
from jax import lax
from jax.experimental import pallas as pl
from jax.experimental.pallas import tpu as pltpu
from jax.experimental.pallas import tpu_sc as plsc

N = 10000
E = 160000
D = 256
DH = 128
NC = 2
NS = 16
NPAD = 10240
DW = 16

_MESH = plsc.VectorSubcoreMesh(
    core_axis_name="c", subcore_axis_name="s", num_cores=NC, num_subcores=NS
)


DEG_B = 40
DEG_EPT = E // (NC * NS)
DEG_NB = DEG_EPT // DEG_B


@functools.partial(
    pl.kernel,
    out_type=jax.ShapeDtypeStruct((NC, N, DW), jnp.float32),
    mesh=_MESH,
    scratch_types=[
        pltpu.VMEM((DEG_B,), jnp.int32),
        pltpu.VMEM((DEG_B,), jnp.int32),
        pltpu.VMEM((DEG_B, DW), jnp.float32),
        pltpu.VMEM((16, DW), jnp.float32),
        pltpu.VMEM_SHARED((NPAD, DW), jnp.float32),
        pltpu.SemaphoreType.DMA,
        pltpu.SemaphoreType.DMA,
    ],
)
def _deg_kernel(row_hbm, degp_hbm, ridx0, ridx1, ones_v, zbuf, acc, sem0, sem1):
    cid = lax.axis_index("c")
    sid = lax.axis_index("s")
    wid = cid * NS + sid
    ebase = wid * DEG_EPT

    def fill_ones(i, _):
        ones_v[i, :] = jnp.ones((DW,), jnp.float32)
        return 0

    lax.fori_loop(0, DEG_B, fill_ones, 0)

    def fill_zero(i, _):
        zbuf[i, :] = jnp.zeros((DW,), jnp.float32)
        return 0

    lax.fori_loop(0, 16, fill_zero, 0)

    def zero_acc(t, _):
        pltpu.sync_copy(zbuf, acc.at[pl.ds(sid * 640 + t * 16, 16)])
        return 0

    lax.fori_loop(0, 40, zero_acc, 0)
    plsc.subcore_barrier()

    pltpu.async_copy(row_hbm.at[pl.ds(ebase, DEG_B)], ridx0, sem0)

    def step(k, ridxc, semc, ridxn, semn):
        pltpu.make_async_copy(row_hbm.at[pl.ds(0, DEG_B)], ridxc, semc).wait()

        @pl.when(k + 1 < DEG_NB)
        def _():
            pltpu.async_copy(
                row_hbm.at[pl.ds(ebase + (k + 1) * DEG_B, DEG_B)], ridxn, semn
            )

        pltpu.sync_copy(ones_v, acc.at[ridxc], add=True)

    def body(k, _):
        @pl.when(k % 2 == 0)
        def _():
            step(k, ridx0, sem0, ridx1, sem1)

        @pl.when(k % 2 == 1)
        def _():
            step(k, ridx1, sem1, ridx0, sem0)

        return 0

    lax.fori_loop(0, DEG_NB, body, 0)
    plsc.subcore_barrier()

    rows = N // NS
    pltpu.sync_copy(
        acc.at[pl.ds(sid * rows, rows)], degp_hbm.at[cid, pl.ds(sid * rows, rows)]
    )


AGG_B = 80
AGG_EPT = E // NS
AGG_NB = AGG_EPT // AGG_B


@functools.partial(
    pl.kernel,
    out_type=jax.ShapeDtypeStruct((NC * N, DH), jnp.float32),
    mesh=_MESH,
    scratch_types=[
        pltpu.VMEM((AGG_B, DH), jnp.float32),
        pltpu.VMEM((AGG_B, DH), jnp.float32),
        pltpu.VMEM((AGG_B,), jnp.int32),
        pltpu.VMEM((AGG_B,), jnp.int32),
        pltpu.VMEM((AGG_B,), jnp.int32),
        pltpu.VMEM((AGG_B,), jnp.int32),
        pltpu.VMEM((16, DH), jnp.float32),
        pltpu.VMEM_SHARED((NPAD, DH), jnp.float32),
        pltpu.SemaphoreType.DMA,
        pltpu.SemaphoreType.DMA,
    ],
)
def _agg_kernel(
    row_hbm, col_hbm, g_hbm, s_hbm,
    buf0, buf1, ridx0, ridx1, cidx0, cidx1, zbuf, acc, sem0, sem1,
):
    cid = lax.axis_index("c")
    sid = lax.axis_index("s")
    ebase = sid * AGG_EPT
    goff = cid * N

    def fill_zero(i, _):
        def inner(j, _):
            zbuf[i, pl.ds(j * 16, 16)] = jnp.zeros((16,), jnp.float32)
            return 0

        lax.fori_loop(0, DH // 16, inner, 0)
        return 0

    lax.fori_loop(0, 16, fill_zero, 0)

    def zero_acc(t, _):
        pltpu.sync_copy(zbuf, acc.at[pl.ds(sid * 640 + t * 16, 16)])
        return 0

    lax.fori_loop(0, 40, zero_acc, 0)
    plsc.subcore_barrier()

    def load_batch(k, ridx, cidx, buf, sem):
        base = ebase + k * AGG_B
        pltpu.sync_copy(row_hbm.at[pl.ds(base, AGG_B)], ridx)

        def adj(i, _):
            ridx[pl.ds(i * 16, 16)] = ridx[pl.ds(i * 16, 16)] + goff
            return 0

        lax.fori_loop(0, AGG_B // 16, adj, 0)
        pltpu.sync_copy(col_hbm.at[pl.ds(base, AGG_B)], cidx)
        pltpu.async_copy(g_hbm.at[ridx], buf, sem)

    load_batch(0, ridx0, cidx0, buf0, sem0)

    def step(k, ridxc, cidxc, bufc, semc, ridxn, cidxn, bufn, semn):
        pltpu.make_async_copy(g_hbm.at[pl.ds(0, AGG_B)], bufc, semc).wait()

        @pl.when(k + 1 < AGG_NB)
        def _():
            load_batch(k + 1, ridxn, cidxn, bufn, semn)

        pltpu.sync_copy(bufc, acc.at[cidxc], add=True)

    def body(k, _):
        @pl.when(k % 2 == 0)
        def _():
            step(k, ridx0, cidx0, buf0, sem0, ridx1, cidx1, buf1, sem1)

        @pl.when(k % 2 == 1)
        def _():
            step(k, ridx1, cidx1, buf1, sem1, ridx0, cidx0, buf0, sem0)

        return 0

    lax.fori_loop(0, AGG_NB, body, 0)
    plsc.subcore_barrier()

    rows = N // NS
    pltpu.sync_copy(
        acc.at[pl.ds(sid * rows, rows)],
        s_hbm.at[pl.ds(goff + sid * rows, rows)],
    )


RBLK = 500
NRB = N // RBLK
_DN = (((1,), (1,)), ((), ()))


def _k1_body(x_ref, w_ref, b_ref, d_ref, o_ref):
    h = lax.dot_general(
        x_ref[...], w_ref[...], _DN, preferred_element_type=jnp.float32
    )
    deg = d_ref[0, :, 0:1] + d_ref[1, :, 0:1] + 1.0
    dis = lax.rsqrt(deg)
    o_ref[...] = dis * (h + b_ref[...])


def _k1(x, W1, b1r, degp):
    return pl.pallas_call(
        _k1_body,
        grid=(NRB, NC),
        in_specs=[
            pl.BlockSpec((RBLK, D), lambda i, j: (i, 0)),
            pl.BlockSpec((DH, D), lambda i, j: (j, 0)),
            pl.BlockSpec((1, DH), lambda i, j: (j, 0)),
            pl.BlockSpec((NC, RBLK, DW), lambda i, j: (0, i, 0)),
        ],
        out_specs=pl.BlockSpec((RBLK, DH), lambda i, j: (j * NRB + i, 0)),
        out_shape=jax.ShapeDtypeStruct((NC * N, DH), jnp.float32),
    )(x, W1, b1r, degp)


def _k2_body(slo_ref, shi_ref, glo_ref, ghi_ref, d_ref, w_ref, b_ref, o_ref):
    deg = d_ref[0, :, 0:1] + d_ref[1, :, 0:1] + 1.0
    dis = lax.rsqrt(deg)
    tlo = jnp.maximum(dis * (slo_ref[...] + glo_ref[...]), 0.0)
    thi = jnp.maximum(dis * (shi_ref[...] + ghi_ref[...]), 0.0)
    t = jnp.concatenate([tlo, thi], axis=1)
    h = lax.dot_general(t, w_ref[...], _DN, preferred_element_type=jnp.float32)
    o_ref[...] = dis * (h + b_ref[...])


def _k2(s1, g1, degp, W2, b2r):
    lo = pl.BlockSpec((RBLK, DH), lambda i, j: (i, 0))
    hi = pl.BlockSpec((RBLK, DH), lambda i, j: (NRB + i, 0))
    return pl.pallas_call(
        _k2_body,
        grid=(NRB, NC),
        in_specs=[
            lo, hi, lo, hi,
            pl.BlockSpec((NC, RBLK, DW), lambda i, j: (0, i, 0)),
            pl.BlockSpec((DH, D), lambda i, j: (j, 0)),
            pl.BlockSpec((1, DH), lambda i, j: (j, 0)),
        ],
        out_specs=pl.BlockSpec((RBLK, DH), lambda i, j: (j * NRB + i, 0)),
        out_shape=jax.ShapeDtypeStruct((NC * N, DH), jnp.float32),
    )(s1, s1, g1, g1, degp, W2, b2r)


def _k3_body(slo_ref, shi_ref, glo_ref, ghi_ref, d_ref, o_ref):
    deg = d_ref[0, :, 0:1] + d_ref[1, :, 0:1] + 1.0
    dis = lax.rsqrt(deg)
    lo = slo_ref[...] + glo_ref[...]
    hi = shi_ref[...] + ghi_ref[...]
    o_ref[...] = dis * jnp.concatenate([lo, hi], axis=1)


def _k3(s2, g2, degp):
    lo = pl.BlockSpec((RBLK, DH), lambda i: (i, 0))
    hi = pl.BlockSpec((RBLK, DH), lambda i: (NRB + i, 0))
    return pl.pallas_call(
        _k3_body,
        grid=(NRB,),
        in_specs=[
            lo, hi, lo, hi,
            pl.BlockSpec((NC, RBLK, DW), lambda i: (0, i, 0)),
        ],
        out_specs=pl.BlockSpec((RBLK, D), lambda i: (i, 0)),
        out_shape=jax.ShapeDtypeStruct((N, D), jnp.float32),
    )(s2, s2, g2, g2, degp)


def kernel(x, edge_index, W1, b1, W2, b2):
    row = edge_index[:, 0].astype(jnp.int32)
    col = edge_index[:, 1].astype(jnp.int32)
    degp = _deg_kernel(row)
    g1 = _k1(x, W1, b1.reshape(NC, DH), degp)
    s1 = _agg_kernel(row, col, g1)
    g2 = _k2(s1, g1, degp, W2, b2.reshape(NC, DH))
    s2 = _agg_kernel(row, col, g2)
    return _k3(s2, g2, degp)


# speedup vs baseline: 6.1443x; 6.1443x over previous
import functools

import jax
import jax.numpy as jnp
from jax import lax
from jax.experimental import pallas as pl
from jax.experimental.pallas import tpu as pltpu
from jax.experimental.pallas import tpu_sc as plsc

N = 10000
E = 160000
D = 256
DH = 128
NC = 2
NS = 16
NPAD = 10240
DW = 16

_MESH = plsc.VectorSubcoreMesh(
    core_axis_name="c", subcore_axis_name="s", num_cores=NC, num_subcores=NS
)


DEG_B = 40
DEG_EPT = E // (NC * NS)
DEG_NB = DEG_EPT // DEG_B


@functools.partial(
    pl.kernel,
    out_type=jax.ShapeDtypeStruct((NC, NPAD, DW), jnp.float32),
    mesh=_MESH,
    scratch_types=[
        pltpu.VMEM((DEG_B,), jnp.int32),
        pltpu.VMEM((DEG_B,), jnp.int32),
        pltpu.VMEM((DEG_B, DW), jnp.float32),
        pltpu.VMEM((16, DW), jnp.float32),
        pltpu.VMEM_SHARED((NPAD, DW), jnp.float32),
        pltpu.SemaphoreType.DMA,
        pltpu.SemaphoreType.DMA,
    ],
)
def _deg_kernel(row_hbm, degp_hbm, ridx0, ridx1, ones_v, zbuf, acc, sem0, sem1):
    cid = lax.axis_index("c")
    sid = lax.axis_index("s")
    wid = cid * NS + sid
    ebase = wid * DEG_EPT

    def fill_ones(i, _):
        ones_v[i, :] = jnp.ones((DW,), jnp.float32)
        return 0

    lax.fori_loop(0, DEG_B, fill_ones, 0)

    def fill_zero(i, _):
        zbuf[i, :] = jnp.zeros((DW,), jnp.float32)
        return 0

    lax.fori_loop(0, 16, fill_zero, 0)

    def zero_acc(t, _):
        pltpu.sync_copy(zbuf, acc.at[pl.ds(sid * 640 + t * 16, 16)])
        return 0

    lax.fori_loop(0, 40, zero_acc, 0)
    plsc.subcore_barrier()

    pltpu.async_copy(row_hbm.at[pl.ds(ebase, DEG_B)], ridx0, sem0)

    def step(k, ridxc, semc, ridxn, semn):
        pltpu.make_async_copy(row_hbm.at[pl.ds(0, DEG_B)], ridxc, semc).wait()

        @pl.when(k + 1 < DEG_NB)
        def _():
            pltpu.async_copy(
                row_hbm.at[pl.ds(ebase + (k + 1) * DEG_B, DEG_B)], ridxn, semn
            )

        pltpu.sync_copy(ones_v, acc.at[ridxc], add=True)

    def body(k, _):
        @pl.when(k % 2 == 0)
        def _():
            step(k, ridx0, sem0, ridx1, sem1)

        @pl.when(k % 2 == 1)
        def _():
            step(k, ridx1, sem1, ridx0, sem0)

        return 0

    lax.fori_loop(0, DEG_NB, body, 0)
    plsc.subcore_barrier()

    pltpu.sync_copy(
        acc.at[pl.ds(sid * 640, 640)], degp_hbm.at[cid, pl.ds(sid * 640, 640)]
    )


AGG_B = 80
AGG_EPT = E // NS
AGG_NB = AGG_EPT // AGG_B


@functools.partial(
    pl.kernel,
    out_type=jax.ShapeDtypeStruct((NC * NPAD, DH), jnp.float32),
    mesh=_MESH,
    scratch_types=[
        pltpu.VMEM((AGG_B, DH), jnp.float32),
        pltpu.VMEM((AGG_B, DH), jnp.float32),
        pltpu.VMEM((AGG_B,), jnp.int32),
        pltpu.VMEM((AGG_B,), jnp.int32),
        pltpu.VMEM((AGG_B,), jnp.int32),
        pltpu.VMEM((AGG_B,), jnp.int32),
        pltpu.VMEM((16, DH), jnp.float32),
        pltpu.VMEM_SHARED((NPAD, DH), jnp.float32),
        pltpu.SemaphoreType.DMA,
        pltpu.SemaphoreType.DMA,
    ],
)
def _agg_kernel(
    row_hbm, col_hbm, g_hbm, s_hbm,
    buf0, buf1, ridx0, ridx1, cidx0, cidx1, zbuf, acc, sem0, sem1,
):
    cid = lax.axis_index("c")
    sid = lax.axis_index("s")
    ebase = sid * AGG_EPT
    goff = cid * NPAD

    def fill_zero(i, _):
        def inner(j, _):
            zbuf[i, pl.ds(j * 16, 16)] = jnp.zeros((16,), jnp.float32)
            return 0

        lax.fori_loop(0, DH // 16, inner, 0)
        return 0

    lax.fori_loop(0, 16, fill_zero, 0)

    def zero_acc(t, _):
        pltpu.sync_copy(zbuf, acc.at[pl.ds(sid * 640 + t * 16, 16)])
        return 0

    lax.fori_loop(0, 40, zero_acc, 0)
    plsc.subcore_barrier()

    def load_batch(k, ridx, cidx, buf, sem):
        base = ebase + k * AGG_B
        pltpu.sync_copy(row_hbm.at[pl.ds(base, AGG_B)], ridx)

        def adj(i, _):
            ridx[pl.ds(i * 16, 16)] = ridx[pl.ds(i * 16, 16)] + goff
            return 0

        lax.fori_loop(0, AGG_B // 16, adj, 0)
        pltpu.sync_copy(col_hbm.at[pl.ds(base, AGG_B)], cidx)
        pltpu.async_copy(g_hbm.at[ridx], buf, sem)

    load_batch(0, ridx0, cidx0, buf0, sem0)

    def step(k, ridxc, cidxc, bufc, semc, ridxn, cidxn, bufn, semn):
        pltpu.make_async_copy(g_hbm.at[pl.ds(0, AGG_B)], bufc, semc).wait()

        @pl.when(k + 1 < AGG_NB)
        def _():
            load_batch(k + 1, ridxn, cidxn, bufn, semn)

        pltpu.sync_copy(bufc, acc.at[cidxc], add=True)

    def body(k, _):
        @pl.when(k % 2 == 0)
        def _():
            step(k, ridx0, cidx0, buf0, sem0, ridx1, cidx1, buf1, sem1)

        @pl.when(k % 2 == 1)
        def _():
            step(k, ridx1, cidx1, buf1, sem1, ridx0, cidx0, buf0, sem0)

        return 0

    lax.fori_loop(0, AGG_NB, body, 0)
    plsc.subcore_barrier()

    pltpu.sync_copy(
        acc.at[pl.ds(sid * 640, 640)],
        s_hbm.at[pl.ds(goff + sid * 640, 640)],
    )


RBLK = 80
NRB = N // RBLK
NPB = NPAD // RBLK
_DN = (((1,), (1,)), ((), ()))


def _k1_body(x_ref, w_ref, b_ref, d_ref, o_ref):
    h = lax.dot_general(
        x_ref[...], w_ref[...], _DN, preferred_element_type=jnp.float32
    )
    deg = d_ref[0, :, 0:1] + d_ref[1, :, 0:1] + 1.0
    dis = lax.rsqrt(deg)
    o_ref[...] = dis * (h + b_ref[0])


def _k1(x, W1, b1r, degp):
    return pl.pallas_call(
        _k1_body,
        grid=(NC, NRB),
        in_specs=[
            pl.BlockSpec((RBLK, D), lambda j, i: (i, 0)),
            pl.BlockSpec((DH, D), lambda j, i: (j, 0)),
            pl.BlockSpec((1, 1, DH), lambda j, i: (j, 0, 0)),
            pl.BlockSpec((NC, RBLK, DW), lambda j, i: (0, i, 0)),
        ],
        out_specs=pl.BlockSpec((RBLK, DH), lambda j, i: (j * NPB + i, 0)),
        out_shape=jax.ShapeDtypeStruct((NC * NPAD, DH), jnp.float32),
    )(x, W1, b1r, degp)


def _k2_body(slo_ref, shi_ref, glo_ref, ghi_ref, d_ref, w_ref, b_ref, o_ref):
    deg = d_ref[0, :, 0:1] + d_ref[1, :, 0:1] + 1.0
    dis = lax.rsqrt(deg)
    tlo = jnp.maximum(dis * (slo_ref[...] + glo_ref[...]), 0.0)
    thi = jnp.maximum(dis * (shi_ref[...] + ghi_ref[...]), 0.0)
    t = jnp.concatenate([tlo, thi], axis=1)
    h = lax.dot_general(t, w_ref[...], _DN, preferred_element_type=jnp.float32)
    o_ref[...] = dis * (h + b_ref[0])


def _k2(s1, g1, degp, W2, b2r):
    lo = pl.BlockSpec((RBLK, DH), lambda j, i: (i, 0))
    hi = pl.BlockSpec((RBLK, DH), lambda j, i: (NPB + i, 0))
    return pl.pallas_call(
        _k2_body,
        grid=(NC, NRB),
        in_specs=[
            lo, hi, lo, hi,
            pl.BlockSpec((NC, RBLK, DW), lambda j, i: (0, i, 0)),
            pl.BlockSpec((DH, D), lambda j, i: (j, 0)),
            pl.BlockSpec((1, 1, DH), lambda j, i: (j, 0, 0)),
        ],
        out_specs=pl.BlockSpec((RBLK, DH), lambda j, i: (j * NPB + i, 0)),
        out_shape=jax.ShapeDtypeStruct((NC * NPAD, DH), jnp.float32),
    )(s1, s1, g1, g1, degp, W2, b2r)


def _k3_body(slo_ref, shi_ref, glo_ref, ghi_ref, d_ref, o_ref):
    deg = d_ref[0, :, 0:1] + d_ref[1, :, 0:1] + 1.0
    dis = lax.rsqrt(deg)
    lo = slo_ref[...] + glo_ref[...]
    hi = shi_ref[...] + ghi_ref[...]
    o_ref[...] = dis * jnp.concatenate([lo, hi], axis=1)


def _k3(s2, g2, degp):
    lo = pl.BlockSpec((RBLK, DH), lambda i: (i, 0))
    hi = pl.BlockSpec((RBLK, DH), lambda i: (NPB + i, 0))
    return pl.pallas_call(
        _k3_body,
        grid=(NRB,),
        in_specs=[
            lo, hi, lo, hi,
            pl.BlockSpec((NC, RBLK, DW), lambda i: (0, i, 0)),
        ],
        out_specs=pl.BlockSpec((RBLK, D), lambda i: (i, 0)),
        out_shape=jax.ShapeDtypeStruct((N, D), jnp.float32),
    )(s2, s2, g2, g2, degp)


def kernel(x, edge_index, W1, b1, W2, b2):
    row = edge_index[:, 0].astype(jnp.int32)
    col = edge_index[:, 1].astype(jnp.int32)
    degp = _deg_kernel(row)
    g1 = _k1(x, W1, b1.reshape(NC, 1, DH), degp)
    s1 = _agg_kernel(row, col, g1)
    g2 = _k2(s1, g1, degp, W2, b2.reshape(NC, 1, DH))
    s2 = _agg_kernel(row, col, g2)
    return _k3(s2, g2, degp)


# speedup vs baseline: 14.1113x; 2.2966x over previous
import functools

import jax
import jax.numpy as jnp
from jax import lax
from jax.experimental import pallas as pl
from jax.experimental.pallas import tpu as pltpu
from jax.experimental.pallas import tpu_sc as plsc

N = 10000
E = 160000
D = 256
DH = 128
NC = 2
NS = 16
NPAD = 10240
HS = 12800
DW = 16

_MESH = plsc.VectorSubcoreMesh(
    core_axis_name="c", subcore_axis_name="s", num_cores=NC, num_subcores=NS
)


DEG_B = 125
DEG_NB = 40


@functools.partial(
    pl.kernel,
    out_type=jax.ShapeDtypeStruct((NC, NPAD, DW), jnp.float32),
    mesh=_MESH,
    scratch_types=[
        pltpu.VMEM((DEG_NB, DEG_B), jnp.int32),
        pltpu.VMEM((DEG_B, DW), jnp.float32),
        pltpu.VMEM((16, DW), jnp.float32),
        pltpu.VMEM_SHARED((NPAD, DW), jnp.float32),
        pltpu.SemaphoreType.DMA,
    ],
)
def _deg_kernel(row_hbm, degp_hbm, ridx, ones_v, zbuf, acc, sem):
    cid = lax.axis_index("c")
    sid = lax.axis_index("s")
    wid = cid * NS + sid

    pltpu.async_copy(row_hbm.at[wid], ridx, sem)

    def fill_ones(i, _):
        ones_v[i, :] = jnp.ones((DW,), jnp.float32)
        return 0

    lax.fori_loop(0, DEG_B, fill_ones, 0)

    def fill_zero(i, _):
        zbuf[i, :] = jnp.zeros((DW,), jnp.float32)
        return 0

    lax.fori_loop(0, 16, fill_zero, 0)

    def zero_acc(t, _):
        pltpu.sync_copy(zbuf, acc.at[pl.ds(sid * 640 + t * 16, 16)])
        return 0

    lax.fori_loop(0, 40, zero_acc, 0)
    pltpu.make_async_copy(row_hbm.at[0], ridx, sem).wait()
    plsc.subcore_barrier()

    def body(k, _):
        pltpu.sync_copy(ones_v, acc.at[ridx.at[k]], add=True)
        return 0

    lax.fori_loop(0, DEG_NB, body, 0)
    plsc.subcore_barrier()

    pltpu.sync_copy(
        acc.at[pl.ds(sid * 640, 640)], degp_hbm.at[cid, pl.ds(sid * 640, 640)]
    )


AGG_B = 80
AGG_NB = 125


@functools.partial(
    pl.kernel,
    out_type=jax.ShapeDtypeStruct((NC * HS, DH), jnp.float32),
    mesh=_MESH,
    scratch_types=[
        pltpu.VMEM((AGG_B, DH), jnp.float32),
        pltpu.VMEM((AGG_B, DH), jnp.float32),
        pltpu.VMEM((AGG_NB * AGG_B,), jnp.int32),
        pltpu.VMEM((AGG_NB, AGG_B), jnp.int32),
        pltpu.VMEM((16, DH), jnp.float32),
        pltpu.VMEM_SHARED((NPAD, DH), jnp.float32),
        pltpu.SemaphoreType.DMA,
        pltpu.SemaphoreType.DMA,
        pltpu.SemaphoreType.DMA,
    ],
)
def _agg_kernel(
    row_hbm, col_hbm, g_hbm, s_hbm,
    buf0, buf1, ridx, cidx, zbuf, acc, sem0, sem1, semi,
):
    cid = lax.axis_index("c")
    sid = lax.axis_index("s")
    goff = cid * HS

    pltpu.async_copy(row_hbm.at[sid], ridx, semi)
    pltpu.async_copy(col_hbm.at[sid], cidx, semi)

    def fill_zero(i, _):
        def inner(j, _):
            zbuf[i, pl.ds(j * 16, 16)] = jnp.zeros((16,), jnp.float32)
            return 0

        lax.fori_loop(0, DH // 16, inner, 0)
        return 0

    lax.fori_loop(0, 16, fill_zero, 0)

    def zero_acc(t, _):
        pltpu.sync_copy(zbuf, acc.at[pl.ds(sid * 640 + t * 16, 16)])
        return 0

    lax.fori_loop(0, 40, zero_acc, 0)

    pltpu.make_async_copy(row_hbm.at[0], ridx, semi).wait()
    pltpu.make_async_copy(col_hbm.at[0], cidx, semi).wait()

    def adj(i, _):
        ridx[pl.ds(i * 16, 16)] = ridx[pl.ds(i * 16, 16)] + goff
        return 0

    lax.fori_loop(0, (AGG_NB * AGG_B) // 16, adj, 0)
    plsc.subcore_barrier()

    def start_gather(k, buf, sem):
        pltpu.async_copy(g_hbm.at[ridx.at[pl.ds(k * AGG_B, AGG_B)]], buf, sem)

    start_gather(0, buf0, sem0)

    def step(k, bufc, semc, bufn, semn):
        pltpu.make_async_copy(g_hbm.at[pl.ds(0, AGG_B)], bufc, semc).wait()

        @pl.when(k + 1 < AGG_NB)
        def _():
            start_gather(k + 1, bufn, semn)

        pltpu.sync_copy(bufc, acc.at[cidx.at[k]], add=True)

    def body(k, _):
        @pl.when(k % 2 == 0)
        def _():
            step(k, buf0, sem0, buf1, sem1)

        @pl.when(k % 2 == 1)
        def _():
            step(k, buf1, sem1, buf0, sem0)

        return 0

    lax.fori_loop(0, AGG_NB, body, 0)
    plsc.subcore_barrier()

    pltpu.sync_copy(
        acc.at[pl.ds(sid * 640, 640)],
        s_hbm.at[pl.ds(goff + sid * 640, 640)],
    )


RBLK = 400
NRB = N // RBLK
_DN = (((1,), (1,)), ((), ()))


def _dis(d_ref):
    deg = d_ref[0, :, 0:1] + d_ref[1, :, 0:1] + 1.0
    return lax.rsqrt(deg)


def _k1_body(x_ref, w_ref, b_ref, d_ref, o_ref):
    h = lax.dot_general(
        x_ref[...], w_ref[...], _DN, preferred_element_type=jnp.float32
    )
    dis = _dis(d_ref)
    o_ref[0] = dis * (h[:, :DH] + b_ref[0])
    o_ref[1] = dis * (h[:, DH:] + b_ref[1])


def _k1(x, W1, b1r, degp):
    return pl.pallas_call(
        _k1_body,
        grid=(NRB,),
        in_specs=[
            pl.BlockSpec((RBLK, D), lambda i: (i, 0)),
            pl.BlockSpec((D, D), lambda i: (0, 0)),
            pl.BlockSpec((NC, 1, DH), lambda i: (0, 0, 0)),
            pl.BlockSpec((NC, RBLK, DW), lambda i: (0, i, 0)),
        ],
        out_specs=pl.BlockSpec((NC, RBLK, DH), lambda i: (0, i, 0)),
        out_shape=jax.ShapeDtypeStruct((NC, HS, DH), jnp.float32),
    )(x, W1, b1r, degp)


def _k2_body(s_ref, g_ref, d_ref, w_ref, b_ref, o_ref):
    dis = _dis(d_ref)
    tlo = jnp.maximum(dis * (s_ref[0] + g_ref[0]), 0.0)
    thi = jnp.maximum(dis * (s_ref[1] + g_ref[1]), 0.0)
    t = jnp.concatenate([tlo, thi], axis=1)
    h = lax.dot_general(t, w_ref[...], _DN, preferred_element_type=jnp.float32)
    o_ref[0] = dis * (h[:, :DH] + b_ref[0])
    o_ref[1] = dis * (h[:, DH:] + b_ref[1])


def _k2(s1, g1, degp, W2, b2r):
    half = pl.BlockSpec((NC, RBLK, DH), lambda i: (0, i, 0))
    return pl.pallas_call(
        _k2_body,
        grid=(NRB,),
        in_specs=[
            half, half,
            pl.BlockSpec((NC, RBLK, DW), lambda i: (0, i, 0)),
            pl.BlockSpec((D, D), lambda i: (0, 0)),
            pl.BlockSpec((NC, 1, DH), lambda i: (0, 0, 0)),
        ],
        out_specs=pl.BlockSpec((NC, RBLK, DH), lambda i: (0, i, 0)),
        out_shape=jax.ShapeDtypeStruct((NC, HS, DH), jnp.float32),
    )(s1, g1, degp, W2, b2r)


def _k3_body(s_ref, g_ref, d_ref, o_ref):
    dis = _dis(d_ref)
    lo = s_ref[0] + g_ref[0]
    hi = s_ref[1] + g_ref[1]
    o_ref[...] = dis * jnp.concatenate([lo, hi], axis=1)


def _k3(s2, g2, degp):
    half = pl.BlockSpec((NC, RBLK, DH), lambda i: (0, i, 0))
    return pl.pallas_call(
        _k3_body,
        grid=(NRB,),
        in_specs=[
            half, half,
            pl.BlockSpec((NC, RBLK, DW), lambda i: (0, i, 0)),
        ],
        out_specs=pl.BlockSpec((RBLK, D), lambda i: (i, 0)),
        out_shape=jax.ShapeDtypeStruct((N, D), jnp.float32),
    )(s2, g2, degp)


def kernel(x, edge_index, W1, b1, W2, b2):
    row = edge_index[:, 0].astype(jnp.int32)
    col = edge_index[:, 1].astype(jnp.int32)
    row_a = row.reshape(NS, AGG_NB * AGG_B)
    col_a = col.reshape(NS, AGG_NB, AGG_B)
    degp = _deg_kernel(row.reshape(NC * NS, DEG_NB, DEG_B))
    g1 = _k1(x, W1, b1.reshape(NC, 1, DH), degp)
    s1 = _agg_kernel(row_a, col_a, g1.reshape(NC * HS, DH)).reshape(NC, HS, DH)
    g2 = _k2(s1, g1, degp, W2, b2.reshape(NC, 1, DH))
    s2 = _agg_kernel(row_a, col_a, g2.reshape(NC * HS, DH)).reshape(NC, HS, DH)
    return _k3(s2, g2, degp)


# speedup vs baseline: 19.2469x; 1.3639x over previous
import functools

import jax
import jax.numpy as jnp
from jax import lax
from jax.experimental import pallas as pl
from jax.experimental.pallas import tpu as pltpu
from jax.experimental.pallas import tpu_sc as plsc

N = 10000
E = 160000
D = 256
DH = 128
NC = 2
NS = 16
NPAD = 10240
HS = 12800
DW = 16

_MESH = plsc.VectorSubcoreMesh(
    core_axis_name="c", subcore_axis_name="s", num_cores=NC, num_subcores=NS
)


DEG_B = 125
DEG_NB = 40


@functools.partial(
    pl.kernel,
    out_type=jax.ShapeDtypeStruct((NC, NPAD, DW), jnp.float32),
    mesh=_MESH,
    scratch_types=[
        pltpu.VMEM((DEG_NB, DEG_B), jnp.int32),
        pltpu.VMEM((DEG_B, DW), jnp.float32),
        pltpu.VMEM((16, DW), jnp.float32),
        pltpu.VMEM_SHARED((NPAD, DW), jnp.float32),
        pltpu.SemaphoreType.DMA,
    ],
)
def _deg_kernel(row_hbm, degp_hbm, ridx, ones_v, zbuf, acc, sem):
    cid = lax.axis_index("c")
    sid = lax.axis_index("s")
    wid = cid * NS + sid

    pltpu.async_copy(row_hbm.at[wid], ridx, sem)

    def fill_ones(i, _):
        ones_v[i, :] = jnp.ones((DW,), jnp.float32)
        return 0

    lax.fori_loop(0, DEG_B, fill_ones, 0)

    def fill_zero(i, _):
        zbuf[i, :] = jnp.zeros((DW,), jnp.float32)
        return 0

    lax.fori_loop(0, 16, fill_zero, 0)

    def zero_acc(t, _):
        pltpu.sync_copy(zbuf, acc.at[pl.ds(sid * 640 + t * 16, 16)])
        return 0

    lax.fori_loop(0, 40, zero_acc, 0)
    pltpu.make_async_copy(row_hbm.at[0], ridx, sem).wait()
    plsc.subcore_barrier()

    def body(k, _):
        pltpu.sync_copy(ones_v, acc.at[ridx.at[k]], add=True)
        return 0

    lax.fori_loop(0, DEG_NB, body, 0)
    plsc.subcore_barrier()

    pltpu.sync_copy(
        acc.at[pl.ds(sid * 640, 640)], degp_hbm.at[cid, pl.ds(sid * 640, 640)]
    )


AGG_B = 80
AGG_NB = 125


@functools.partial(
    pl.kernel,
    out_type=jax.ShapeDtypeStruct((NC * HS, DH), jnp.float32),
    mesh=_MESH,
    scratch_types=[
        pltpu.VMEM((AGG_B, DH), jnp.float32),
        pltpu.VMEM((AGG_B, DH), jnp.float32),
        pltpu.VMEM((AGG_B, DH), jnp.float32),
        pltpu.VMEM((AGG_B,), jnp.int32),
        pltpu.VMEM((AGG_B,), jnp.int32),
        pltpu.VMEM((AGG_B,), jnp.int32),
        pltpu.VMEM((AGG_NB, AGG_B), jnp.int32),
        pltpu.VMEM((8, DH), jnp.float32),
        pltpu.VMEM_SHARED((NS * 632, DH), jnp.float32),
        pltpu.SemaphoreType.DMA,
        pltpu.SemaphoreType.DMA,
        pltpu.SemaphoreType.DMA,
        pltpu.SemaphoreType.DMA,
        pltpu.SemaphoreType.DMA,
        pltpu.SemaphoreType.DMA,
        pltpu.SemaphoreType.DMA,
        pltpu.SemaphoreType.DMA,
        pltpu.SemaphoreType.DMA,
    ],
)
def _agg_kernel(
    row_hbm, col_hbm, g_hbm, s_hbm,
    buf0, buf1, buf2, r0, r1, r2, cidx, zbuf, acc,
    sg0, sg1, sg2, ss0, ss1, ss2, si0, si1, si2,
):
    cid = lax.axis_index("c")
    sid = lax.axis_index("s")
    goff = cid * HS
    ebase = sid * AGG_NB * AGG_B

    pltpu.async_copy(col_hbm.at[sid], cidx, si0)

    def fill_zero(i, _):
        def inner(j, _):
            zbuf[i, pl.ds(j * 16, 16)] = jnp.zeros((16,), jnp.float32)
            return 0

        lax.fori_loop(0, DH // 16, inner, 0)
        return 0

    lax.fori_loop(0, 8, fill_zero, 0)

    def zero_acc(t, _):
        pltpu.sync_copy(zbuf, acc.at[pl.ds(sid * 632 + t * 8, 8)])
        return 0

    lax.fori_loop(0, 79, zero_acc, 0)
    pltpu.make_async_copy(col_hbm.at[0], cidx, si0).wait()
    plsc.subcore_barrier()

    bufs = (buf0, buf1, buf2)
    ridxs = (r0, r1, r2)
    sgs = (sg0, sg1, sg2)
    sss = (ss0, ss1, ss2)
    sis = (si0, si1, si2)

    def start_idx(k, r, sem):
        pltpu.async_copy(row_hbm.at[pl.ds(ebase + k * AGG_B, AGG_B)], r, sem)

    def adjust(r):
        def adj(i, _):
            r[pl.ds(i * 16, 16)] = r[pl.ds(i * 16, 16)] + goff
            return 0

        lax.fori_loop(0, AGG_B // 16, adj, 0)

    def start_gather(r, buf, sem):
        pltpu.async_copy(g_hbm.at[r], buf, sem)

    def gdrain(buf, sem):
        pltpu.make_async_copy(g_hbm.at[pl.ds(0, AGG_B)], buf, sem).wait()

    def idrain(r, sem):
        pltpu.make_async_copy(row_hbm.at[pl.ds(0, AGG_B)], r, sem).wait()

    for j in range(2):
        start_idx(j, ridxs[j], sis[j])
        idrain(ridxs[j], sis[j])
        adjust(ridxs[j])
        start_gather(ridxs[j], bufs[j], sgs[j])
    start_idx(2, r2, si2)

    def step(k, p):
        q = (p + 2) % 3
        gdrain(bufs[p], sgs[p])
        pltpu.async_copy(bufs[p], acc.at[cidx.at[k]], sss[p], add=True)

        @pl.when(k + 3 < AGG_NB)
        def _():
            start_idx(k + 3, ridxs[p], sis[p])

        @pl.when(k + 2 < AGG_NB)
        def _():
            idrain(ridxs[q], sis[q])
            adjust(ridxs[q])

            @pl.when(k >= 1)
            def _():
                gdrain(bufs[q], sss[q])

            start_gather(ridxs[q], bufs[q], sgs[q])

    def body(k, _):
        for p in range(3):
            @pl.when(k % 3 == p)
            def _(p=p):
                step(k, p)

        return 0

    lax.fori_loop(0, AGG_NB, body, 0)
    for p in range(3):
        gdrain(bufs[p], sss[p])
    plsc.subcore_barrier()

    pltpu.sync_copy(
        acc.at[pl.ds(sid * 632, 632)],
        s_hbm.at[pl.ds(goff + sid * 632, 632)],
    )


RBLK = 400
NRB = N // RBLK
_DN = (((1,), (1,)), ((), ()))


def _dis(d_ref):
    deg = d_ref[0, :, 0:1] + d_ref[1, :, 0:1] + 1.0
    return lax.rsqrt(deg)


def _k1_body(x_ref, w_ref, b_ref, d_ref, o_ref):
    h = lax.dot_general(
        x_ref[...], w_ref[...], _DN, preferred_element_type=jnp.float32
    )
    dis = _dis(d_ref)
    o_ref[0] = dis * (h[:, :DH] + b_ref[0])
    o_ref[1] = dis * (h[:, DH:] + b_ref[1])


def _k1(x, W1, b1r, degp):
    return pl.pallas_call(
        _k1_body,
        grid=(NRB,),
        in_specs=[
            pl.BlockSpec((RBLK, D), lambda i: (i, 0)),
            pl.BlockSpec((D, D), lambda i: (0, 0)),
            pl.BlockSpec((NC, 1, DH), lambda i: (0, 0, 0)),
            pl.BlockSpec((NC, RBLK, DW), lambda i: (0, i, 0)),
        ],
        out_specs=pl.BlockSpec((NC, RBLK, DH), lambda i: (0, i, 0)),
        out_shape=jax.ShapeDtypeStruct((NC, HS, DH), jnp.float32),
    )(x, W1, b1r, degp)


def _k2_body(s_ref, g_ref, d_ref, w_ref, b_ref, o_ref):
    dis = _dis(d_ref)
    tlo = jnp.maximum(dis * (s_ref[0] + g_ref[0]), 0.0)
    thi = jnp.maximum(dis * (s_ref[1] + g_ref[1]), 0.0)
    t = jnp.concatenate([tlo, thi], axis=1)
    h = lax.dot_general(t, w_ref[...], _DN, preferred_element_type=jnp.float32)
    o_ref[0] = dis * (h[:, :DH] + b_ref[0])
    o_ref[1] = dis * (h[:, DH:] + b_ref[1])


def _k2(s1, g1, degp, W2, b2r):
    half = pl.BlockSpec((NC, RBLK, DH), lambda i: (0, i, 0))
    return pl.pallas_call(
        _k2_body,
        grid=(NRB,),
        in_specs=[
            half, half,
            pl.BlockSpec((NC, RBLK, DW), lambda i: (0, i, 0)),
            pl.BlockSpec((D, D), lambda i: (0, 0)),
            pl.BlockSpec((NC, 1, DH), lambda i: (0, 0, 0)),
        ],
        out_specs=pl.BlockSpec((NC, RBLK, DH), lambda i: (0, i, 0)),
        out_shape=jax.ShapeDtypeStruct((NC, HS, DH), jnp.float32),
    )(s1, g1, degp, W2, b2r)


def _k3_body(s_ref, g_ref, d_ref, o_ref):
    dis = _dis(d_ref)
    lo = s_ref[0] + g_ref[0]
    hi = s_ref[1] + g_ref[1]
    o_ref[...] = dis * jnp.concatenate([lo, hi], axis=1)


def _k3(s2, g2, degp):
    half = pl.BlockSpec((NC, RBLK, DH), lambda i: (0, i, 0))
    return pl.pallas_call(
        _k3_body,
        grid=(NRB,),
        in_specs=[
            half, half,
            pl.BlockSpec((NC, RBLK, DW), lambda i: (0, i, 0)),
        ],
        out_specs=pl.BlockSpec((RBLK, D), lambda i: (i, 0)),
        out_shape=jax.ShapeDtypeStruct((N, D), jnp.float32),
    )(s2, g2, degp)


def kernel(x, edge_index, W1, b1, W2, b2):
    row = edge_index[:, 0].astype(jnp.int32)
    col = edge_index[:, 1].astype(jnp.int32)
    row_a = row
    col_a = col.reshape(NS, AGG_NB, AGG_B)
    degp = _deg_kernel(row.reshape(NC * NS, DEG_NB, DEG_B))
    g1 = _k1(x, W1, b1.reshape(NC, 1, DH), degp)
    s1 = _agg_kernel(row_a, col_a, g1.reshape(NC * HS, DH)).reshape(NC, HS, DH)
    g2 = _k2(s1, g1, degp, W2, b2.reshape(NC, 1, DH))
    s2 = _agg_kernel(row_a, col_a, g2.reshape(NC * HS, DH)).reshape(NC, HS, DH)
    return _k3(s2, g2, degp)
